# baseline (speedup 1.0000x reference)
import jax
import jax.numpy as jnp
from jax.experimental import pallas as pl
from jax.experimental.pallas import tpu as pltpu

_ROWS = 128
_CHUNK = 8
_NBUF = 16
_NCHUNKS = _ROWS // _CHUNK


def _masked_softmax_rows(x, m_raw):
    m = m_raw == 1
    neg = jnp.finfo(x.dtype).min
    z = jnp.where(m, x, neg)
    mx = jnp.max(z, axis=-1, keepdims=True)
    e = jnp.exp(z - mx)
    s = jnp.sum(e, axis=-1, keepdims=True)
    scale = jnp.where(
        mx > neg,
        jnp.asarray(1.0, x.dtype) / jnp.maximum(s, jnp.asarray(1e-30, x.dtype)),
        jnp.zeros((), x.dtype),
    )
    return e * scale


def _pipelined_kernel(x_hbm, m_hbm, o_hbm, xb, mb, ob, in_sem, out_sem):
    def in_copies(i, slot):
        rows = pl.ds(i * _CHUNK, _CHUNK)
        return (
            pltpu.make_async_copy(x_hbm.at[rows], xb.at[slot], in_sem.at[slot, 0]),
            pltpu.make_async_copy(m_hbm.at[rows], mb.at[slot], in_sem.at[slot, 1]),
        )

    def out_copy(i, slot):
        rows = pl.ds(i * _CHUNK, _CHUNK)
        return pltpu.make_async_copy(ob.at[slot], o_hbm.at[rows], out_sem.at[slot])

    for i in range(_NBUF):
        cx, cm = in_copies(i, i)
        cx.start()
        cm.start()

    for i in range(_NCHUNKS):
        slot = i % _NBUF
        cx, cm = in_copies(i, slot)
        cx.wait()
        cm.wait()
        if i >= _NBUF:
            out_copy(i - _NBUF, slot).wait()
        ob[slot] = _masked_softmax_rows(xb[slot], mb[slot])
        out_copy(i, slot).start()
        nxt = i + _NBUF
        if nxt < _NCHUNKS:
            nx, nm = in_copies(nxt, slot)
            nx.start()
            nm.start()

    for i in range(_NCHUNKS - _NBUF, _NCHUNKS):
        out_copy(i, i % _NBUF).wait()


def kernel(inputLayer, mask):
    rows, cols = inputLayer.shape
    any_spec = pl.BlockSpec(memory_space=pl.ANY)
    return pl.pallas_call(
        _pipelined_kernel,
        in_specs=[any_spec, any_spec],
        out_specs=any_spec,
        out_shape=jax.ShapeDtypeStruct((rows, cols), inputLayer.dtype),
        scratch_shapes=[
            pltpu.VMEM((_NBUF, _CHUNK, cols), jnp.float32),
            pltpu.VMEM((_NBUF, _CHUNK, cols), jnp.int32),
            pltpu.VMEM((_NBUF, _CHUNK, cols), jnp.float32),
            pltpu.SemaphoreType.DMA((_NBUF, 2)),
            pltpu.SemaphoreType.DMA((_NBUF,)),
        ],
    )(inputLayer, mask)


# speedup vs baseline: 1.0129x; 1.0129x over previous
import jax
import jax.numpy as jnp
from jax.experimental import pallas as pl
from jax.experimental.pallas import tpu as pltpu

_ROWS = 128
_CHUNK = 8
_NBUF = 8
_NCHUNKS = _ROWS // _CHUNK


def _masked_softmax_rows(x, m_raw):
    m = m_raw == 1
    neg = jnp.finfo(x.dtype).min
    z = jnp.where(m, x, neg)
    mx = jnp.max(z, axis=-1, keepdims=True)
    e = jnp.exp(z - mx)
    s = jnp.sum(e, axis=-1, keepdims=True)
    scale = jnp.where(
        mx > neg,
        jnp.asarray(1.0, x.dtype) / jnp.maximum(s, jnp.asarray(1e-30, x.dtype)),
        jnp.zeros((), x.dtype),
    )
    return e * scale


def _pipelined_kernel(x_hbm, m_hbm, o_hbm, xb, mb, ob, in_sem, out_sem):
    def in_copies(i, slot):
        rows = pl.ds(i * _CHUNK, _CHUNK)
        return (
            pltpu.make_async_copy(x_hbm.at[rows], xb.at[slot], in_sem.at[slot, 0]),
            pltpu.make_async_copy(m_hbm.at[rows], mb.at[slot], in_sem.at[slot, 1]),
        )

    def out_copy(i, slot):
        rows = pl.ds(i * _CHUNK, _CHUNK)
        return pltpu.make_async_copy(ob.at[slot], o_hbm.at[rows], out_sem.at[slot])

    for i in range(_NBUF):
        cx, cm = in_copies(i, i)
        cx.start()
        cm.start()

    for i in range(_NCHUNKS):
        slot = i % _NBUF
        cx, cm = in_copies(i, slot)
        cx.wait()
        cm.wait()
        if i >= _NBUF:
            out_copy(i - _NBUF, slot).wait()
        ob[slot] = _masked_softmax_rows(xb[slot], mb[slot])
        out_copy(i, slot).start()
        nxt = i + _NBUF
        if nxt < _NCHUNKS:
            nx, nm = in_copies(nxt, slot)
            nx.start()
            nm.start()

    for i in range(_NCHUNKS - _NBUF, _NCHUNKS):
        out_copy(i, i % _NBUF).wait()


def kernel(inputLayer, mask):
    rows, cols = inputLayer.shape
    any_spec = pl.BlockSpec(memory_space=pl.ANY)
    return pl.pallas_call(
        _pipelined_kernel,
        in_specs=[any_spec, any_spec],
        out_specs=any_spec,
        out_shape=jax.ShapeDtypeStruct((rows, cols), inputLayer.dtype),
        scratch_shapes=[
            pltpu.VMEM((_NBUF, _CHUNK, cols), jnp.float32),
            pltpu.VMEM((_NBUF, _CHUNK, cols), jnp.int32),
            pltpu.VMEM((_NBUF, _CHUNK, cols), jnp.float32),
            pltpu.SemaphoreType.DMA((_NBUF, 2)),
            pltpu.SemaphoreType.DMA((_NBUF,)),
        ],
    )(inputLayer, mask)


# speedup vs baseline: 1.0197x; 1.0068x over previous
import jax
import jax.numpy as jnp
from jax.experimental import pallas as pl
from jax.experimental.pallas import tpu as pltpu

_ROWS = 128
_CHUNK = 8
_NBUF = 6
_NCHUNKS = _ROWS // _CHUNK


def _masked_softmax_rows(x, m_raw):
    m = m_raw == 1
    neg = jnp.finfo(x.dtype).min
    z = jnp.where(m, x, neg)
    mx = jnp.max(z, axis=-1, keepdims=True)
    e = jnp.exp(z - mx)
    s = jnp.sum(e, axis=-1, keepdims=True)
    scale = jnp.where(
        mx > neg,
        jnp.asarray(1.0, x.dtype) / jnp.maximum(s, jnp.asarray(1e-30, x.dtype)),
        jnp.zeros((), x.dtype),
    )
    return e * scale


def _pipelined_kernel(x_hbm, m_hbm, o_hbm, xb, mb, ob, in_sem, out_sem):
    def in_copies(i, slot):
        rows = pl.ds(i * _CHUNK, _CHUNK)
        return (
            pltpu.make_async_copy(x_hbm.at[rows], xb.at[slot], in_sem.at[slot, 0]),
            pltpu.make_async_copy(m_hbm.at[rows], mb.at[slot], in_sem.at[slot, 1]),
        )

    def out_copy(i, slot):
        rows = pl.ds(i * _CHUNK, _CHUNK)
        return pltpu.make_async_copy(ob.at[slot], o_hbm.at[rows], out_sem.at[slot])

    for i in range(_NBUF):
        cx, cm = in_copies(i, i)
        cx.start()
        cm.start()

    for i in range(_NCHUNKS):
        slot = i % _NBUF
        cx, cm = in_copies(i, slot)
        cx.wait()
        cm.wait()
        if i >= _NBUF:
            out_copy(i - _NBUF, slot).wait()
        ob[slot] = _masked_softmax_rows(xb[slot], mb[slot])
        out_copy(i, slot).start()
        nxt = i + _NBUF
        if nxt < _NCHUNKS:
            nx, nm = in_copies(nxt, slot)
            nx.start()
            nm.start()

    for i in range(_NCHUNKS - _NBUF, _NCHUNKS):
        out_copy(i, i % _NBUF).wait()


def kernel(inputLayer, mask):
    rows, cols = inputLayer.shape
    any_spec = pl.BlockSpec(memory_space=pl.ANY)
    return pl.pallas_call(
        _pipelined_kernel,
        in_specs=[any_spec, any_spec],
        out_specs=any_spec,
        out_shape=jax.ShapeDtypeStruct((rows, cols), inputLayer.dtype),
        scratch_shapes=[
            pltpu.VMEM((_NBUF, _CHUNK, cols), jnp.float32),
            pltpu.VMEM((_NBUF, _CHUNK, cols), jnp.int32),
            pltpu.VMEM((_NBUF, _CHUNK, cols), jnp.float32),
            pltpu.SemaphoreType.DMA((_NBUF, 2)),
            pltpu.SemaphoreType.DMA((_NBUF,)),
        ],
    )(inputLayer, mask)
